# Initial kernel scaffold; baseline (speedup 1.0000x reference)
import jax
import jax.numpy as jnp
from jax.experimental import pallas as pl


def kernel(x, w_router, w_up, w_down, w_proj):
    raise NotImplementedError("write your pallas kernel here")


# speedup vs baseline: 3.4456x; 3.4456x over previous
import functools

import jax
import jax.numpy as jnp
from jax import lax
from jax.experimental import pallas as pl
from jax.experimental.pallas import tpu as pltpu
from jax.experimental.pallas import tpu_sc as plsc

NE = 8
S = 2048
D = 768
DFF2 = 4096
VPAD = 1024
TM = 128
NT = 40
NTPAD = 64
PAD = NT * TM
NP = 2 * S

_NC, _NS = 2, 16
NW = _NC * _NS
TW = S // NW


def _router_kernel(x_ref, wr_ref, cw0_ref, cw1_ref, dk0_ref, dk1_ref,
                   te_ref, tv_ref):
    x = x_ref[...]
    scores = lax.dot_general(x, wr_ref[...], (((1,), (1,)), ((), ())),
                             preferred_element_type=jnp.float32)
    col = lax.broadcasted_iota(jnp.int32, scores.shape, 1)
    neg = jnp.float32(-3e38)
    sm = jnp.where(col < NE, scores, neg)
    m1 = jnp.max(sm, axis=1, keepdims=True)
    i1 = jnp.min(jnp.where(sm == m1, col, 9999), axis=1, keepdims=True)
    s2 = jnp.where(col == i1, neg, sm)
    m2 = jnp.max(s2, axis=1, keepdims=True)
    i2 = jnp.min(jnp.where(s2 == m2, col, 9999), axis=1, keepdims=True)
    cw0 = 1.0 / (1.0 + jnp.exp(m2 - m1))
    cw0_ref[...] = cw0
    cw1_ref[...] = 1.0 - cw0

    oh1 = (col == i1).astype(jnp.float32)
    oh2 = (col == i2).astype(jnp.float32)
    cnt = oh1 + oh2
    r = lax.broadcasted_iota(jnp.int32, (TM, TM), 0)
    c = lax.broadcasted_iota(jnp.int32, (TM, TM), 1)
    tri = (r > c).astype(jnp.bfloat16)
    blocks = []
    carry = jnp.zeros((1, 128), jnp.float32)
    for b in range(S // TM):
        cb = cnt[b * TM:(b + 1) * TM]
        blocks.append(lax.dot_general(tri, cb.astype(jnp.bfloat16),
                                      (((1,), (0,)), ((), ())),
                                      preferred_element_type=jnp.float32)
                      + carry)
        carry = carry + jnp.sum(cb, axis=0, keepdims=True)
    C = jnp.concatenate(blocks, axis=0)
    counts = carry
    rank1 = jnp.sum(C * oh1, axis=1, keepdims=True)
    rank2 = jnp.sum(C * oh2, axis=1, keepdims=True)

    ci = counts.astype(jnp.int32)
    pc = ((ci + TM - 1) // TM) * TM
    er = lax.broadcasted_iota(jnp.int32, (128, 128), 0)
    ec = lax.broadcasted_iota(jnp.int32, (128, 128), 1)
    triL = (er < ec).astype(jnp.float32)
    poff = lax.dot_general(pc.astype(jnp.float32), triL,
                           (((1,), (0,)), ((), ())),
                           preferred_element_type=jnp.float32)
    dk0 = jnp.sum(oh1 * poff, axis=1, keepdims=True) + rank1
    dk1 = jnp.sum(oh2 * poff, axis=1, keepdims=True) + rank2
    dk0_ref[...] = dk0.astype(jnp.int32)
    dk1_ref[...] = dk1.astype(jnp.int32)

    ts = lax.broadcasted_iota(jnp.int32, (NTPAD, 128), 0) * TM
    poffb = poff.astype(jnp.int32)
    pcb = pc
    inr = ((poffb <= ts) & (ts < poffb + pcb)).astype(jnp.int32)
    ecol = lax.broadcasted_iota(jnp.int32, (NTPAD, 128), 1)
    te = jnp.sum(inr * ecol, axis=1, keepdims=True)
    cnt_sel = jnp.sum(inr * ci, axis=1, keepdims=True)
    poff_sel = jnp.sum(inr * poffb, axis=1, keepdims=True)
    tstart = lax.broadcasted_iota(jnp.int32, (NTPAD, 1), 0) * TM
    tv = jnp.clip(cnt_sel - (tstart - poff_sel), 0, TM)
    te_ref[...] = te
    tv_ref[...] = tv


def _run_router(x2, wrp):
    f32 = jnp.float32
    i32 = jnp.int32
    return pl.pallas_call(
        _router_kernel,
        out_shape=(
            jax.ShapeDtypeStruct((S, 1), f32),
            jax.ShapeDtypeStruct((S, 1), f32),
            jax.ShapeDtypeStruct((S, 1), i32),
            jax.ShapeDtypeStruct((S, 1), i32),
            jax.ShapeDtypeStruct((NTPAD, 1), i32),
            jax.ShapeDtypeStruct((NTPAD, 1), i32),
        ),
    )(x2, wrp)


def _dispatch_scatter(x2, dk0, dk1):
    mesh = plsc.VectorSubcoreMesh(core_axis_name="c", subcore_axis_name="s")

    @functools.partial(
        pl.kernel,
        mesh=mesh,
        out_type=jax.ShapeDtypeStruct((PAD, D), jnp.float32),
        scratch_types=[
            pltpu.VMEM((TW, D), jnp.float32),
            pltpu.VMEM((TW,), jnp.int32),
            pltpu.VMEM((TW,), jnp.int32),
            pltpu.SemaphoreType.DMA,
        ],
    )
    def body(x_hbm, dk0_hbm, dk1_hbm, buf_hbm, rows_v, idx0_v, idx1_v, sem):
        wid = lax.axis_index("s") * _NC + lax.axis_index("c")
        base = wid * TW
        pltpu.sync_copy(x_hbm.at[pl.ds(base, TW)], rows_v)
        pltpu.sync_copy(dk0_hbm.at[pl.ds(base, TW)], idx0_v)
        pltpu.sync_copy(dk1_hbm.at[pl.ds(base, TW)], idx1_v)
        pltpu.async_copy(rows_v, buf_hbm.at[idx0_v], sem).wait()
        pltpu.async_copy(rows_v, buf_hbm.at[idx1_v], sem).wait()

    return body(x2, dk0, dk1)


def _gmm_kernel(te_ref, tv_ref, x_ref, wup_ref, wdn_ref, out_ref):
    t = pl.program_id(0)
    v = tv_ref[t]
    x = x_ref[...]
    rows = lax.broadcasted_iota(jnp.int32, (TM, 1), 0)
    x = jnp.where(rows < v, x, 0.0)
    xb = x.astype(jnp.bfloat16)
    h1 = lax.dot_general(xb, wup_ref[0], (((1,), (1,)), ((), ())),
                         preferred_element_type=jnp.float32)
    h1 = h1.astype(jnp.bfloat16).astype(jnp.float32)
    a = h1[:, :DFF2 // 2]
    g = h1[:, DFF2 // 2:]
    sw = a * (g * (1.0 / (1.0 + jnp.exp(-g))))
    h2 = lax.dot_general(sw.astype(jnp.bfloat16), wdn_ref[0],
                         (((1,), (1,)), ((), ())),
                         preferred_element_type=jnp.float32)
    h2 = h2.astype(jnp.bfloat16).astype(jnp.float32)
    out_ref[...] = h2 + x


def _run_gmm(buf, wup_bf, wdn_bf, te, tv):
    grid_spec = pltpu.PrefetchScalarGridSpec(
        num_scalar_prefetch=2,
        grid=(NT,),
        in_specs=[
            pl.BlockSpec((TM, D), lambda t, te, tv: (t, 0)),
            pl.BlockSpec((1, DFF2, D), lambda t, te, tv: (te[t], 0, 0)),
            pl.BlockSpec((1, D, DFF2 // 2), lambda t, te, tv: (te[t], 0, 0)),
        ],
        out_specs=pl.BlockSpec((TM, D), lambda t, te, tv: (t, 0)),
    )
    return pl.pallas_call(
        _gmm_kernel,
        grid_spec=grid_spec,
        out_shape=jax.ShapeDtypeStruct((PAD, D), jnp.float32),
    )(te, tv, buf, wup_bf, wdn_bf)


def _combine_gather(h3, dk0, dk1):
    mesh = plsc.VectorSubcoreMesh(core_axis_name="c", subcore_axis_name="s")

    @functools.partial(
        pl.kernel,
        mesh=mesh,
        out_type=(
            jax.ShapeDtypeStruct((S, D), jnp.float32),
            jax.ShapeDtypeStruct((S, D), jnp.float32),
        ),
        scratch_types=[
            pltpu.VMEM((TW, D), jnp.float32),
            pltpu.VMEM((TW,), jnp.int32),
            pltpu.SemaphoreType.DMA,
        ],
    )
    def body(h3_hbm, dk0_hbm, dk1_hbm, g0_hbm, g1_hbm, rows_v, idx_v, sem):
        wid = lax.axis_index("s") * _NC + lax.axis_index("c")
        base = wid * TW
        pltpu.sync_copy(dk0_hbm.at[pl.ds(base, TW)], idx_v)
        pltpu.async_copy(h3_hbm.at[idx_v], rows_v, sem).wait()
        pltpu.sync_copy(rows_v, g0_hbm.at[pl.ds(base, TW)])
        pltpu.sync_copy(dk1_hbm.at[pl.ds(base, TW)], idx_v)
        pltpu.async_copy(h3_hbm.at[idx_v], rows_v, sem).wait()
        pltpu.sync_copy(rows_v, g1_hbm.at[pl.ds(base, TW)])

    return body(h3, dk0, dk1)


def _proj_kernel(g0_ref, g1_ref, cw0_ref, cw1_ref, wp_ref, out_ref):
    o0 = lax.dot_general(g0_ref[...].astype(jnp.bfloat16), wp_ref[...],
                         (((1,), (1,)), ((), ())),
                         preferred_element_type=jnp.float32)
    o0 = o0.astype(jnp.bfloat16).astype(jnp.float32)
    o1 = lax.dot_general(g1_ref[...].astype(jnp.bfloat16), wp_ref[...],
                         (((1,), (1,)), ((), ())),
                         preferred_element_type=jnp.float32)
    o1 = o1.astype(jnp.bfloat16).astype(jnp.float32)
    out_ref[...] = cw0_ref[...] * o0 + cw1_ref[...] * o1


def _run_proj(g0, g1, cw0, cw1, wp_bf):
    TN = 256
    return pl.pallas_call(
        _proj_kernel,
        grid=(S // TN,),
        in_specs=[
            pl.BlockSpec((TN, D), lambda t: (t, 0)),
            pl.BlockSpec((TN, D), lambda t: (t, 0)),
            pl.BlockSpec((TN, 1), lambda t: (t, 0)),
            pl.BlockSpec((TN, 1), lambda t: (t, 0)),
            pl.BlockSpec((VPAD, D), lambda t: (0, 0)),
        ],
        out_specs=pl.BlockSpec((TN, VPAD), lambda t: (t, 0)),
        out_shape=jax.ShapeDtypeStruct((S, VPAD), jnp.float32),
    )(g0, g1, cw0, cw1, wp_bf)


def kernel(x, w_router, w_up, w_down, w_proj):
    B, SS, DD = x.shape
    x2 = x.reshape(SS, DD)
    wrp = jnp.pad(w_router, ((0, 128 - NE), (0, 0)))
    wup_bf = w_up.astype(jnp.bfloat16)
    wdn_bf = w_down.astype(jnp.bfloat16)
    wp_bf = jnp.pad(w_proj[0], ((0, VPAD - w_proj.shape[1]), (0, 0))
                    ).astype(jnp.bfloat16)

    cw0, cw1, dk0, dk1, te, tv = _run_router(x2, wrp)
    buf = _dispatch_scatter(x2, dk0.reshape(S), dk1.reshape(S))
    h3 = _run_gmm(buf, wup_bf, wdn_bf, te.reshape(NTPAD), tv.reshape(NTPAD))
    g0, g1 = _combine_gather(h3, dk0.reshape(S), dk1.reshape(S))
    out = _run_proj(g0, g1, cw0, cw1, wp_bf)
    return out[:, :w_proj.shape[1]].reshape(B, SS, w_proj.shape[1])


# speedup vs baseline: 4.0126x; 1.1645x over previous
import functools

import jax
import jax.numpy as jnp
from jax import lax
from jax.experimental import pallas as pl
from jax.experimental.pallas import tpu as pltpu
from jax.experimental.pallas import tpu_sc as plsc

NE = 8
S = 2048
D = 768
DFF2 = 4096
VPAD = 1024
TM = 128
NT = 40
NTPAD = 64
PAD = NT * TM
NP = 2 * S

_NC, _NS = 2, 16
NW = _NC * _NS
TW = S // NW


def _router_kernel(x_ref, wr_ref, cw0_ref, cw1_ref, dk0_ref, dk1_ref,
                   te_ref, tv_ref):
    x = x_ref[...]
    scores = lax.dot_general(x, wr_ref[...], (((1,), (1,)), ((), ())),
                             preferred_element_type=jnp.float32)
    col = lax.broadcasted_iota(jnp.int32, scores.shape, 1)
    neg = jnp.float32(-3e38)
    sm = jnp.where(col < NE, scores, neg)
    m1 = jnp.max(sm, axis=1, keepdims=True)
    i1 = jnp.min(jnp.where(sm == m1, col, 9999), axis=1, keepdims=True)
    s2 = jnp.where(col == i1, neg, sm)
    m2 = jnp.max(s2, axis=1, keepdims=True)
    i2 = jnp.min(jnp.where(s2 == m2, col, 9999), axis=1, keepdims=True)
    cw0 = 1.0 / (1.0 + jnp.exp(m2 - m1))
    cw0_ref[...] = cw0
    cw1_ref[...] = 1.0 - cw0

    oh1 = (col == i1).astype(jnp.float32)
    oh2 = (col == i2).astype(jnp.float32)
    cnt = oh1 + oh2
    r = lax.broadcasted_iota(jnp.int32, (TM, TM), 0)
    c = lax.broadcasted_iota(jnp.int32, (TM, TM), 1)
    tri = (r > c).astype(jnp.bfloat16)
    blocks = []
    carry = jnp.zeros((1, 128), jnp.float32)
    for b in range(S // TM):
        cb = cnt[b * TM:(b + 1) * TM]
        blocks.append(lax.dot_general(tri, cb.astype(jnp.bfloat16),
                                      (((1,), (0,)), ((), ())),
                                      preferred_element_type=jnp.float32)
                      + carry)
        carry = carry + jnp.sum(cb, axis=0, keepdims=True)
    C = jnp.concatenate(blocks, axis=0)
    counts = carry
    rank1 = jnp.sum(C * oh1, axis=1, keepdims=True)
    rank2 = jnp.sum(C * oh2, axis=1, keepdims=True)

    ci = counts.astype(jnp.int32)
    pc = ((ci + TM - 1) // TM) * TM
    er = lax.broadcasted_iota(jnp.int32, (128, 128), 0)
    ec = lax.broadcasted_iota(jnp.int32, (128, 128), 1)
    triL = (er < ec).astype(jnp.float32)
    poff = lax.dot_general(pc.astype(jnp.float32), triL,
                           (((1,), (0,)), ((), ())),
                           preferred_element_type=jnp.float32)
    dk0 = jnp.sum(oh1 * poff, axis=1, keepdims=True) + rank1
    dk1 = jnp.sum(oh2 * poff, axis=1, keepdims=True) + rank2
    dk0_ref[...] = dk0.astype(jnp.int32)
    dk1_ref[...] = dk1.astype(jnp.int32)

    ts = lax.broadcasted_iota(jnp.int32, (NTPAD, 128), 0) * TM
    poffb = poff.astype(jnp.int32)
    pcb = pc
    inr = ((poffb <= ts) & (ts < poffb + pcb)).astype(jnp.int32)
    ecol = lax.broadcasted_iota(jnp.int32, (NTPAD, 128), 1)
    te = jnp.sum(inr * ecol, axis=1, keepdims=True)
    cnt_sel = jnp.sum(inr * ci, axis=1, keepdims=True)
    poff_sel = jnp.sum(inr * poffb, axis=1, keepdims=True)
    tstart = lax.broadcasted_iota(jnp.int32, (NTPAD, 1), 0) * TM
    tv = jnp.clip(cnt_sel - (tstart - poff_sel), 0, TM)
    te_ref[...] = te
    tv_ref[...] = tv


def _run_router(x2, wrp):
    f32 = jnp.float32
    i32 = jnp.int32
    return pl.pallas_call(
        _router_kernel,
        out_shape=(
            jax.ShapeDtypeStruct((S, 1), f32),
            jax.ShapeDtypeStruct((S, 1), f32),
            jax.ShapeDtypeStruct((S, 1), i32),
            jax.ShapeDtypeStruct((S, 1), i32),
            jax.ShapeDtypeStruct((NTPAD, 1), i32),
            jax.ShapeDtypeStruct((NTPAD, 1), i32),
        ),
    )(x2, wrp)


def _dispatch_scatter(x2, dk0, dk1):
    mesh = plsc.VectorSubcoreMesh(core_axis_name="c", subcore_axis_name="s")

    @functools.partial(
        pl.kernel,
        mesh=mesh,
        out_type=jax.ShapeDtypeStruct((PAD, D), jnp.float32),
        scratch_types=[
            pltpu.VMEM((TW, D), jnp.float32),
            pltpu.VMEM((TW,), jnp.int32),
            pltpu.VMEM((TW,), jnp.int32),
            pltpu.SemaphoreType.DMA,
        ],
    )
    def body(x_hbm, dk0_hbm, dk1_hbm, buf_hbm, rows_v, idx0_v, idx1_v, sem):
        wid = lax.axis_index("s") * _NC + lax.axis_index("c")
        base = wid * TW
        pltpu.sync_copy(x_hbm.at[pl.ds(base, TW)], rows_v)
        pltpu.sync_copy(dk0_hbm.at[pl.ds(base, TW)], idx0_v)
        pltpu.sync_copy(dk1_hbm.at[pl.ds(base, TW)], idx1_v)
        pltpu.async_copy(rows_v, buf_hbm.at[idx0_v], sem).wait()
        pltpu.async_copy(rows_v, buf_hbm.at[idx1_v], sem).wait()

    return body(x2, dk0, dk1)


def _gmm_kernel(te_ref, tv_ref, x_ref, wup_ref, wdn_ref, out_ref,
                wupb_ref, wdnb_ref):
    t = pl.program_id(0)
    e = te_ref[t]
    eprev = te_ref[jnp.maximum(t - 1, 0)]

    @pl.when((t == 0) | (e != eprev))
    def _cast():
        wupb_ref[...] = wup_ref[0].astype(jnp.bfloat16)
        wdnb_ref[...] = wdn_ref[0].astype(jnp.bfloat16)

    v = tv_ref[t]
    x = x_ref[...]
    rows = lax.broadcasted_iota(jnp.int32, (TM, 1), 0)
    x = jnp.where(rows < v, x, 0.0)
    xb = x.astype(jnp.bfloat16)
    h1 = lax.dot_general(xb, wupb_ref[...], (((1,), (1,)), ((), ())),
                         preferred_element_type=jnp.float32)
    h1 = h1.astype(jnp.bfloat16).astype(jnp.float32)
    a = h1[:, :DFF2 // 2]
    g = h1[:, DFF2 // 2:]
    sw = a * (g * (1.0 / (1.0 + jnp.exp(-g))))
    h2 = lax.dot_general(sw.astype(jnp.bfloat16), wdnb_ref[...],
                         (((1,), (1,)), ((), ())),
                         preferred_element_type=jnp.float32)
    h2 = h2.astype(jnp.bfloat16).astype(jnp.float32)
    out_ref[...] = h2 + x


def _run_gmm(buf, w_up, w_down, te, tv):
    grid_spec = pltpu.PrefetchScalarGridSpec(
        num_scalar_prefetch=2,
        grid=(NT,),
        in_specs=[
            pl.BlockSpec((TM, D), lambda t, te, tv: (t, 0)),
            pl.BlockSpec((1, DFF2, D), lambda t, te, tv: (te[t], 0, 0)),
            pl.BlockSpec((1, D, DFF2 // 2), lambda t, te, tv: (te[t], 0, 0)),
        ],
        out_specs=pl.BlockSpec((TM, D), lambda t, te, tv: (t, 0)),
        scratch_shapes=[
            pltpu.VMEM((DFF2, D), jnp.bfloat16),
            pltpu.VMEM((D, DFF2 // 2), jnp.bfloat16),
        ],
    )
    return pl.pallas_call(
        _gmm_kernel,
        grid_spec=grid_spec,
        out_shape=jax.ShapeDtypeStruct((PAD, D), jnp.float32),
    )(te, tv, buf, w_up, w_down)


def _combine_gather(h3, dk0, dk1):
    mesh = plsc.VectorSubcoreMesh(core_axis_name="c", subcore_axis_name="s")

    @functools.partial(
        pl.kernel,
        mesh=mesh,
        out_type=(
            jax.ShapeDtypeStruct((S, D), jnp.float32),
            jax.ShapeDtypeStruct((S, D), jnp.float32),
        ),
        scratch_types=[
            pltpu.VMEM((TW, D), jnp.float32),
            pltpu.VMEM((TW,), jnp.int32),
            pltpu.SemaphoreType.DMA,
        ],
    )
    def body(h3_hbm, dk0_hbm, dk1_hbm, g0_hbm, g1_hbm, rows_v, idx_v, sem):
        wid = lax.axis_index("s") * _NC + lax.axis_index("c")
        base = wid * TW
        pltpu.sync_copy(dk0_hbm.at[pl.ds(base, TW)], idx_v)
        pltpu.async_copy(h3_hbm.at[idx_v], rows_v, sem).wait()
        pltpu.sync_copy(rows_v, g0_hbm.at[pl.ds(base, TW)])
        pltpu.sync_copy(dk1_hbm.at[pl.ds(base, TW)], idx_v)
        pltpu.async_copy(h3_hbm.at[idx_v], rows_v, sem).wait()
        pltpu.sync_copy(rows_v, g1_hbm.at[pl.ds(base, TW)])

    return body(h3, dk0, dk1)


def _proj_kernel(g0_ref, g1_ref, cw0_ref, cw1_ref, wp_ref, out_ref):
    hc = cw0_ref[...] * g0_ref[...] + cw1_ref[...] * g1_ref[...]
    out_ref[...] = lax.dot_general(hc.astype(jnp.bfloat16),
                                   wp_ref[...].astype(jnp.bfloat16),
                                   (((1,), (1,)), ((), ())),
                                   preferred_element_type=jnp.float32)


def _run_proj(g0, g1, cw0, cw1, wp, vocab):
    TN = 256
    return pl.pallas_call(
        _proj_kernel,
        grid=(S // TN,),
        in_specs=[
            pl.BlockSpec((TN, D), lambda t: (t, 0)),
            pl.BlockSpec((TN, D), lambda t: (t, 0)),
            pl.BlockSpec((TN, 1), lambda t: (t, 0)),
            pl.BlockSpec((TN, 1), lambda t: (t, 0)),
            pl.BlockSpec((vocab, D), lambda t: (0, 0)),
        ],
        out_specs=pl.BlockSpec((TN, vocab), lambda t: (t, 0)),
        out_shape=jax.ShapeDtypeStruct((S, vocab), jnp.float32),
    )(g0, g1, cw0, cw1, wp)


def kernel(x, w_router, w_up, w_down, w_proj):
    B, SS, DD = x.shape
    x2 = x.reshape(SS, DD)
    wrp = jnp.pad(w_router, ((0, 128 - NE), (0, 0)))

    cw0, cw1, dk0, dk1, te, tv = _run_router(x2, wrp)
    buf = _dispatch_scatter(x2, dk0.reshape(S), dk1.reshape(S))
    h3 = _run_gmm(buf, w_up, w_down, te.reshape(NTPAD), tv.reshape(NTPAD))
    g0, g1 = _combine_gather(h3, dk0.reshape(S), dk1.reshape(S))
    out = _run_proj(g0, g1, cw0, cw1, w_proj[0], w_proj.shape[1])
    return out.reshape(B, SS, w_proj.shape[1])


# speedup vs baseline: 4.5200x; 1.1264x over previous
import functools

import jax
import jax.numpy as jnp
from jax import lax
from jax.experimental import pallas as pl
from jax.experimental.pallas import tpu as pltpu
from jax.experimental.pallas import tpu_sc as plsc

NE = 8
S = 2048
D = 768
DFF2 = 4096
VPAD = 1024
TM = 128
NT = 40
NTPAD = 64
PAD = NT * TM
NP = 2 * S

_NC, _NS = 2, 16
NW = _NC * _NS
TW = S // NW


def _router_kernel(x_ref, wr_ref, cw0_ref, cw1_ref, dk0_ref, dk1_ref,
                   te_ref, tv_ref, sl_ref, pf_ref, en_ref, fi_ref):
    x = x_ref[...]
    scores = lax.dot_general(x, wr_ref[...], (((1,), (1,)), ((), ())),
                             preferred_element_type=jnp.float32)
    col = lax.broadcasted_iota(jnp.int32, scores.shape, 1)
    neg = jnp.float32(-3e38)
    sm = jnp.where(col < NE, scores, neg)
    m1 = jnp.max(sm, axis=1, keepdims=True)
    i1 = jnp.min(jnp.where(sm == m1, col, 9999), axis=1, keepdims=True)
    s2 = jnp.where(col == i1, neg, sm)
    m2 = jnp.max(s2, axis=1, keepdims=True)
    i2 = jnp.min(jnp.where(s2 == m2, col, 9999), axis=1, keepdims=True)
    cw0 = 1.0 / (1.0 + jnp.exp(m2 - m1))
    cw0_ref[...] = cw0
    cw1_ref[...] = 1.0 - cw0

    oh1 = (col == i1).astype(jnp.float32)
    oh2 = (col == i2).astype(jnp.float32)
    cnt = oh1 + oh2
    r = lax.broadcasted_iota(jnp.int32, (TM, TM), 0)
    c = lax.broadcasted_iota(jnp.int32, (TM, TM), 1)
    tri = (r > c).astype(jnp.bfloat16)
    blocks = []
    carry = jnp.zeros((1, 128), jnp.float32)
    for b in range(S // TM):
        cb = cnt[b * TM:(b + 1) * TM]
        blocks.append(lax.dot_general(tri, cb.astype(jnp.bfloat16),
                                      (((1,), (0,)), ((), ())),
                                      preferred_element_type=jnp.float32)
                      + carry)
        carry = carry + jnp.sum(cb, axis=0, keepdims=True)
    C = jnp.concatenate(blocks, axis=0)
    counts = carry
    rank1 = jnp.sum(C * oh1, axis=1, keepdims=True)
    rank2 = jnp.sum(C * oh2, axis=1, keepdims=True)

    ci = counts.astype(jnp.int32)
    pc = ((ci + TM - 1) // TM) * TM
    er = lax.broadcasted_iota(jnp.int32, (128, 128), 0)
    ec = lax.broadcasted_iota(jnp.int32, (128, 128), 1)
    triL = (er < ec).astype(jnp.float32)
    poff = lax.dot_general(pc.astype(jnp.float32), triL,
                           (((1,), (0,)), ((), ())),
                           preferred_element_type=jnp.float32)
    dk0 = jnp.sum(oh1 * poff, axis=1, keepdims=True) + rank1
    dk1 = jnp.sum(oh2 * poff, axis=1, keepdims=True) + rank2
    dk0_ref[...] = dk0.astype(jnp.int32)
    dk1_ref[...] = dk1.astype(jnp.int32)

    ts = lax.broadcasted_iota(jnp.int32, (NTPAD, 128), 0) * TM
    poffb = poff.astype(jnp.int32)
    ecol = lax.broadcasted_iota(jnp.int32, (NTPAD, 128), 1)
    used = (pc > 0).astype(jnp.int32)
    started = used * (poffb <= ts)
    k_ord = jnp.maximum(jnp.sum(started, axis=1, keepdims=True) - 1, 0)
    ordmat = lax.dot_general(used.astype(jnp.float32), triL,
                             (((1,), (0,)), ((), ())),
                             preferred_element_type=jnp.float32)
    ord_of = (ordmat.astype(jnp.int32) + used - 1)
    n_used = jnp.sum(used)
    sel_cur = ((ord_of == k_ord) & (used > 0)).astype(jnp.int32)
    te = jnp.sum(sel_cur * ecol, axis=1, keepdims=True)
    cnt_sel = jnp.sum(sel_cur * ci, axis=1, keepdims=True)
    poff_sel = jnp.sum(sel_cur * poffb, axis=1, keepdims=True)
    tstart = lax.broadcasted_iota(jnp.int32, (NTPAD, 1), 0) * TM
    tv = jnp.clip(cnt_sel - (tstart - poff_sel), 0, TM)
    sel_nxt = ((ord_of == k_ord + 1) & (used > 0)).astype(jnp.int32)
    enext = jnp.sum(sel_nxt * ecol, axis=1, keepdims=True)
    first = (tstart == poff_sel).astype(jnp.int32)
    pref = first * (k_ord + 1 < n_used).astype(jnp.int32)
    slot = k_ord & 1
    te_ref[...] = te
    tv_ref[...] = tv
    sl_ref[...] = slot
    pf_ref[...] = pref
    en_ref[...] = enext
    fi_ref[...] = first


def _run_router(x2, wrp):
    f32 = jnp.float32
    i32 = jnp.int32
    return pl.pallas_call(
        _router_kernel,
        out_shape=(
            jax.ShapeDtypeStruct((S, 1), f32),
            jax.ShapeDtypeStruct((S, 1), f32),
            jax.ShapeDtypeStruct((S, 1), i32),
            jax.ShapeDtypeStruct((S, 1), i32),
            jax.ShapeDtypeStruct((NTPAD, 1), i32),
            jax.ShapeDtypeStruct((NTPAD, 1), i32),
            jax.ShapeDtypeStruct((NTPAD, 1), i32),
            jax.ShapeDtypeStruct((NTPAD, 1), i32),
            jax.ShapeDtypeStruct((NTPAD, 1), i32),
            jax.ShapeDtypeStruct((NTPAD, 1), i32),
        ),
    )(x2, wrp)


def _dispatch_scatter(x2, dk0, dk1):
    mesh = plsc.VectorSubcoreMesh(core_axis_name="c", subcore_axis_name="s")

    @functools.partial(
        pl.kernel,
        mesh=mesh,
        out_type=jax.ShapeDtypeStruct((PAD, D), jnp.float32),
        scratch_types=[
            pltpu.VMEM((TW, D), jnp.float32),
            pltpu.VMEM((TW,), jnp.int32),
            pltpu.VMEM((TW,), jnp.int32),
            pltpu.SemaphoreType.DMA,
        ],
    )
    def body(x_hbm, dk0_hbm, dk1_hbm, buf_hbm, rows_v, idx0_v, idx1_v, sem):
        wid = lax.axis_index("s") * _NC + lax.axis_index("c")
        base = wid * TW
        pltpu.sync_copy(x_hbm.at[pl.ds(base, TW)], rows_v)
        pltpu.sync_copy(dk0_hbm.at[pl.ds(base, TW)], idx0_v)
        pltpu.sync_copy(dk1_hbm.at[pl.ds(base, TW)], idx1_v)
        pltpu.async_copy(rows_v, buf_hbm.at[idx0_v], sem).wait()
        pltpu.async_copy(rows_v, buf_hbm.at[idx1_v], sem).wait()

    return body(x2, dk0, dk1)


def _wdma(wup_hbm, wdn_hbm, wup_v, wdn_v, sems, e, s):
    cp_up = pltpu.make_async_copy(wup_hbm.at[e], wup_v.at[s], sems.at[s, 0])
    cp_dn = pltpu.make_async_copy(wdn_hbm.at[e], wdn_v.at[s], sems.at[s, 1])
    return cp_up, cp_dn


def _gmm_kernel(te_ref, tv_ref, sl_ref, pf_ref, en_ref, fi_ref,
                x_ref, wup_hbm, wdn_hbm, out_ref,
                wup_v, wdn_v, wupb_ref, wdnb_ref, sems):
    t = pl.program_id(0)
    e = te_ref[t]
    s = sl_ref[t]
    first = fi_ref[t] == 1

    @pl.when(t == 0)
    def _start_first():
        cu, cd = _wdma(wup_hbm, wdn_hbm, wup_v, wdn_v, sems, e, s)
        cu.start()
        cd.start()

    @pl.when(pf_ref[t] == 1)
    def _start_next():
        cu, cd = _wdma(wup_hbm, wdn_hbm, wup_v, wdn_v, sems,
                       en_ref[t], 1 - s)
        cu.start()
        cd.start()

    @pl.when(first | (t == 0))
    def _wait_and_cast():
        cu, cd = _wdma(wup_hbm, wdn_hbm, wup_v, wdn_v, sems, e, s)
        cu.wait()
        cd.wait()
        wupb_ref[...] = wup_v[s].astype(jnp.bfloat16)
        wdnb_ref[...] = wdn_v[s].astype(jnp.bfloat16)

    v = tv_ref[t]
    x = x_ref[...]
    rows = lax.broadcasted_iota(jnp.int32, (TM, 1), 0)
    x = jnp.where(rows < v, x, 0.0)
    xb = x.astype(jnp.bfloat16)
    h1 = lax.dot_general(xb, wupb_ref[...], (((1,), (1,)), ((), ())),
                         preferred_element_type=jnp.float32)
    h1 = h1.astype(jnp.bfloat16).astype(jnp.float32)
    a = h1[:, :DFF2 // 2]
    g = h1[:, DFF2 // 2:]
    sw = a * (g * (1.0 / (1.0 + jnp.exp(-g))))
    h2 = lax.dot_general(sw.astype(jnp.bfloat16), wdnb_ref[...],
                         (((1,), (1,)), ((), ())),
                         preferred_element_type=jnp.float32)
    h2 = h2.astype(jnp.bfloat16).astype(jnp.float32)
    out_ref[...] = h2 + x


def _run_gmm(buf, w_up, w_down, te, tv, sl, pf, en, fi):
    grid_spec = pltpu.PrefetchScalarGridSpec(
        num_scalar_prefetch=6,
        grid=(NT,),
        in_specs=[
            pl.BlockSpec((TM, D), lambda t, *_: (t, 0)),
            pl.BlockSpec(memory_space=pl.ANY),
            pl.BlockSpec(memory_space=pl.ANY),
        ],
        out_specs=pl.BlockSpec((TM, D), lambda t, *_: (t, 0)),
        scratch_shapes=[
            pltpu.VMEM((2, DFF2, D), jnp.float32),
            pltpu.VMEM((2, D, DFF2 // 2), jnp.float32),
            pltpu.VMEM((DFF2, D), jnp.bfloat16),
            pltpu.VMEM((D, DFF2 // 2), jnp.bfloat16),
            pltpu.SemaphoreType.DMA((2, 2)),
        ],
    )
    return pl.pallas_call(
        _gmm_kernel,
        grid_spec=grid_spec,
        out_shape=jax.ShapeDtypeStruct((PAD, D), jnp.float32),
    )(te, tv, sl, pf, en, fi, buf, w_up, w_down)


def _combine_gather(h3, dk0, dk1):
    mesh = plsc.VectorSubcoreMesh(core_axis_name="c", subcore_axis_name="s")

    @functools.partial(
        pl.kernel,
        mesh=mesh,
        out_type=(
            jax.ShapeDtypeStruct((S, D), jnp.float32),
            jax.ShapeDtypeStruct((S, D), jnp.float32),
        ),
        scratch_types=[
            pltpu.VMEM((TW, D), jnp.float32),
            pltpu.VMEM((TW,), jnp.int32),
            pltpu.SemaphoreType.DMA,
        ],
    )
    def body(h3_hbm, dk0_hbm, dk1_hbm, g0_hbm, g1_hbm, rows_v, idx_v, sem):
        wid = lax.axis_index("s") * _NC + lax.axis_index("c")
        base = wid * TW
        pltpu.sync_copy(dk0_hbm.at[pl.ds(base, TW)], idx_v)
        pltpu.async_copy(h3_hbm.at[idx_v], rows_v, sem).wait()
        pltpu.sync_copy(rows_v, g0_hbm.at[pl.ds(base, TW)])
        pltpu.sync_copy(dk1_hbm.at[pl.ds(base, TW)], idx_v)
        pltpu.async_copy(h3_hbm.at[idx_v], rows_v, sem).wait()
        pltpu.sync_copy(rows_v, g1_hbm.at[pl.ds(base, TW)])

    return body(h3, dk0, dk1)


def _proj_kernel(g0_ref, g1_ref, cw0_ref, cw1_ref, wp_ref, out_ref):
    hc = cw0_ref[...] * g0_ref[...] + cw1_ref[...] * g1_ref[...]
    out_ref[...] = lax.dot_general(hc.astype(jnp.bfloat16),
                                   wp_ref[0].astype(jnp.bfloat16),
                                   (((1,), (1,)), ((), ())),
                                   preferred_element_type=jnp.float32)


def _run_proj(g0, g1, cw0, cw1, w_proj):
    TN = 256
    vocab = w_proj.shape[1]
    return pl.pallas_call(
        _proj_kernel,
        grid=(S // TN,),
        in_specs=[
            pl.BlockSpec((TN, D), lambda t: (t, 0)),
            pl.BlockSpec((TN, D), lambda t: (t, 0)),
            pl.BlockSpec((TN, 1), lambda t: (t, 0)),
            pl.BlockSpec((TN, 1), lambda t: (t, 0)),
            pl.BlockSpec((1, vocab, D), lambda t: (0, 0, 0)),
        ],
        out_specs=pl.BlockSpec((TN, vocab), lambda t: (t, 0)),
        out_shape=jax.ShapeDtypeStruct((S, vocab), jnp.float32),
    )(g0, g1, cw0, cw1, w_proj)


def kernel(x, w_router, w_up, w_down, w_proj):
    B, SS, DD = x.shape
    x2 = x.reshape(SS, DD)
    wrp = jnp.pad(w_router, ((0, 128 - NE), (0, 0)))

    cw0, cw1, dk0, dk1, te, tv, sl, pf, en, fi = _run_router(x2, wrp)
    buf = _dispatch_scatter(x2, dk0.reshape(S), dk1.reshape(S))
    h3 = _run_gmm(buf, w_up, w_down, te.reshape(NTPAD), tv.reshape(NTPAD),
                  sl.reshape(NTPAD), pf.reshape(NTPAD), en.reshape(NTPAD),
                  fi.reshape(NTPAD))
    g0, g1 = _combine_gather(h3, dk0.reshape(S), dk1.reshape(S))
    out = _run_proj(g0, g1, cw0, cw1, w_proj)
    return out.reshape(B, SS, w_proj.shape[1])


# speedup vs baseline: 4.5291x; 1.0020x over previous
import functools

import jax
import jax.numpy as jnp
from jax import lax
from jax.experimental import pallas as pl
from jax.experimental.pallas import tpu as pltpu
from jax.experimental.pallas import tpu_sc as plsc

NE = 8
S = 2048
D = 768
DFF2 = 4096
VPAD = 1024
TM = 128
NT = 40
NTPAD = 64
PAD = NT * TM
NP = 2 * S

_NC, _NS = 2, 16
NW = _NC * _NS
TW = S // NW


def _router_kernel(x_ref, wr_ref, cw0_ref, cw1_ref, dk0_ref, dk1_ref,
                   te_ref, tv_ref, sl_ref, pf_ref, en_ref, fi_ref):
    x = x_ref[...]
    scores = lax.dot_general(x, wr_ref[...], (((1,), (1,)), ((), ())),
                             preferred_element_type=jnp.float32)
    col = lax.broadcasted_iota(jnp.int32, scores.shape, 1)
    neg = jnp.float32(-3e38)
    sm = jnp.where(col < NE, scores, neg)
    m1 = jnp.max(sm, axis=1, keepdims=True)
    i1 = jnp.min(jnp.where(sm == m1, col, 9999), axis=1, keepdims=True)
    s2 = jnp.where(col == i1, neg, sm)
    m2 = jnp.max(s2, axis=1, keepdims=True)
    i2 = jnp.min(jnp.where(s2 == m2, col, 9999), axis=1, keepdims=True)
    cw0 = 1.0 / (1.0 + jnp.exp(m2 - m1))
    cw0_ref[...] = cw0
    cw1_ref[...] = 1.0 - cw0

    oh1 = (col == i1).astype(jnp.float32)
    oh2 = (col == i2).astype(jnp.float32)
    cnt = oh1 + oh2
    r = lax.broadcasted_iota(jnp.int32, (TM, TM), 0)
    c = lax.broadcasted_iota(jnp.int32, (TM, TM), 1)
    tri = (r > c).astype(jnp.bfloat16)
    blocks = []
    carry = jnp.zeros((1, 128), jnp.float32)
    for b in range(S // TM):
        cb = cnt[b * TM:(b + 1) * TM]
        blocks.append(lax.dot_general(tri, cb.astype(jnp.bfloat16),
                                      (((1,), (0,)), ((), ())),
                                      preferred_element_type=jnp.float32)
                      + carry)
        carry = carry + jnp.sum(cb, axis=0, keepdims=True)
    C = jnp.concatenate(blocks, axis=0)
    counts = carry
    rank1 = jnp.sum(C * oh1, axis=1, keepdims=True)
    rank2 = jnp.sum(C * oh2, axis=1, keepdims=True)

    ci = counts.astype(jnp.int32)
    pc = ((ci + TM - 1) // TM) * TM
    er = lax.broadcasted_iota(jnp.int32, (128, 128), 0)
    ec = lax.broadcasted_iota(jnp.int32, (128, 128), 1)
    triL = (er < ec).astype(jnp.float32)
    poff = lax.dot_general(pc.astype(jnp.float32), triL,
                           (((1,), (0,)), ((), ())),
                           preferred_element_type=jnp.float32)
    dk0 = jnp.sum(oh1 * poff, axis=1, keepdims=True) + rank1
    dk1 = jnp.sum(oh2 * poff, axis=1, keepdims=True) + rank2
    dk0_ref[...] = dk0.astype(jnp.int32)
    dk1_ref[...] = dk1.astype(jnp.int32)

    ts = lax.broadcasted_iota(jnp.int32, (NTPAD, 128), 0) * TM
    poffb = poff.astype(jnp.int32)
    ecol = lax.broadcasted_iota(jnp.int32, (NTPAD, 128), 1)
    used = (pc > 0).astype(jnp.int32)
    started = used * (poffb <= ts)
    k_ord = jnp.maximum(jnp.sum(started, axis=1, keepdims=True) - 1, 0)
    ordmat = lax.dot_general(used.astype(jnp.float32), triL,
                             (((1,), (0,)), ((), ())),
                             preferred_element_type=jnp.float32)
    ord_of = (ordmat.astype(jnp.int32) + used - 1)
    n_used = jnp.sum(used)
    sel_cur = ((ord_of == k_ord) & (used > 0)).astype(jnp.int32)
    te = jnp.sum(sel_cur * ecol, axis=1, keepdims=True)
    cnt_sel = jnp.sum(sel_cur * ci, axis=1, keepdims=True)
    poff_sel = jnp.sum(sel_cur * poffb, axis=1, keepdims=True)
    tstart = lax.broadcasted_iota(jnp.int32, (NTPAD, 1), 0) * TM
    tv = jnp.clip(cnt_sel - (tstart - poff_sel), 0, TM)
    sel_nxt = ((ord_of == k_ord + 1) & (used > 0)).astype(jnp.int32)
    enext = jnp.sum(sel_nxt * ecol, axis=1, keepdims=True)
    first = (tstart == poff_sel).astype(jnp.int32)
    pref = first * (k_ord + 1 < n_used).astype(jnp.int32)
    slot = k_ord & 1
    te_ref[...] = te
    tv_ref[...] = tv
    sl_ref[...] = slot
    pf_ref[...] = pref
    en_ref[...] = enext
    fi_ref[...] = first


def _run_router(x2, wrp):
    f32 = jnp.float32
    i32 = jnp.int32
    return pl.pallas_call(
        _router_kernel,
        out_shape=(
            jax.ShapeDtypeStruct((S, 1), f32),
            jax.ShapeDtypeStruct((S, 1), f32),
            jax.ShapeDtypeStruct((S, 1), i32),
            jax.ShapeDtypeStruct((S, 1), i32),
            jax.ShapeDtypeStruct((NTPAD, 1), i32),
            jax.ShapeDtypeStruct((NTPAD, 1), i32),
            jax.ShapeDtypeStruct((NTPAD, 1), i32),
            jax.ShapeDtypeStruct((NTPAD, 1), i32),
            jax.ShapeDtypeStruct((NTPAD, 1), i32),
            jax.ShapeDtypeStruct((NTPAD, 1), i32),
        ),
    )(x2, wrp)


def _dispatch_scatter(x2, dk0, dk1):
    mesh = plsc.VectorSubcoreMesh(core_axis_name="c", subcore_axis_name="s")

    @functools.partial(
        pl.kernel,
        mesh=mesh,
        out_type=jax.ShapeDtypeStruct((PAD, D), jnp.float32),
        scratch_types=[
            pltpu.VMEM((TW, D), jnp.float32),
            pltpu.VMEM((TW,), jnp.int32),
            pltpu.VMEM((TW,), jnp.int32),
            pltpu.SemaphoreType.DMA,
        ],
    )
    def body(x_hbm, dk0_hbm, dk1_hbm, buf_hbm, rows_v, idx0_v, idx1_v, sem):
        wid = lax.axis_index("s") * _NC + lax.axis_index("c")
        base = wid * TW
        pltpu.sync_copy(x_hbm.at[pl.ds(base, TW)], rows_v)
        pltpu.sync_copy(dk0_hbm.at[pl.ds(base, TW)], idx0_v)
        pltpu.sync_copy(dk1_hbm.at[pl.ds(base, TW)], idx1_v)
        pltpu.async_copy(rows_v, buf_hbm.at[idx0_v], sem).wait()
        pltpu.async_copy(rows_v, buf_hbm.at[idx1_v], sem).wait()

    return body(x2, dk0, dk1)


def _wdma(wup_hbm, wdn_hbm, wup_v, wdn_v, sems, e, s):
    h = DFF2 // 2
    q = D // 2
    return (
        pltpu.make_async_copy(wup_hbm.at[e, pl.ds(0, h)],
                              wup_v.at[s, pl.ds(0, h)], sems.at[s, 0]),
        pltpu.make_async_copy(wup_hbm.at[e, pl.ds(h, h)],
                              wup_v.at[s, pl.ds(h, h)], sems.at[s, 1]),
        pltpu.make_async_copy(wdn_hbm.at[e, pl.ds(0, q)],
                              wdn_v.at[s, pl.ds(0, q)], sems.at[s, 2]),
        pltpu.make_async_copy(wdn_hbm.at[e, pl.ds(q, q)],
                              wdn_v.at[s, pl.ds(q, q)], sems.at[s, 3]),
    )


def _gmm_kernel(te_ref, tv_ref, sl_ref, pf_ref, en_ref, fi_ref,
                x_ref, wup_hbm, wdn_hbm, out_ref,
                wup_v, wdn_v, wupb_ref, wdnb_ref, sems):
    t = pl.program_id(0)
    e = te_ref[t]
    s = sl_ref[t]
    first = fi_ref[t] == 1

    @pl.when(t == 0)
    def _start_first():
        for cp in _wdma(wup_hbm, wdn_hbm, wup_v, wdn_v, sems, e, s):
            cp.start()

    @pl.when(pf_ref[t] == 1)
    def _start_next():
        for cp in _wdma(wup_hbm, wdn_hbm, wup_v, wdn_v, sems,
                        en_ref[t], 1 - s):
            cp.start()

    @pl.when(first | (t == 0))
    def _wait_and_cast():
        for cp in _wdma(wup_hbm, wdn_hbm, wup_v, wdn_v, sems, e, s):
            cp.wait()
        wupb_ref[...] = wup_v[s].astype(jnp.bfloat16)
        wdnb_ref[...] = wdn_v[s].astype(jnp.bfloat16)

    v = tv_ref[t]
    x = x_ref[...]
    rows = lax.broadcasted_iota(jnp.int32, (TM, 1), 0)
    x = jnp.where(rows < v, x, 0.0)
    xb = x.astype(jnp.bfloat16)
    h1 = lax.dot_general(xb, wupb_ref[...], (((1,), (1,)), ((), ())),
                         preferred_element_type=jnp.float32)
    h1 = h1.astype(jnp.bfloat16).astype(jnp.float32)
    a = h1[:, :DFF2 // 2]
    g = h1[:, DFF2 // 2:]
    sw = a * (g * (1.0 / (1.0 + jnp.exp(-g))))
    h2 = lax.dot_general(sw.astype(jnp.bfloat16), wdnb_ref[...],
                         (((1,), (1,)), ((), ())),
                         preferred_element_type=jnp.float32)
    h2 = h2.astype(jnp.bfloat16).astype(jnp.float32)
    out_ref[...] = h2 + x


def _run_gmm(buf, w_up, w_down, te, tv, sl, pf, en, fi):
    grid_spec = pltpu.PrefetchScalarGridSpec(
        num_scalar_prefetch=6,
        grid=(NT,),
        in_specs=[
            pl.BlockSpec((TM, D), lambda t, *_: (t, 0)),
            pl.BlockSpec(memory_space=pl.ANY),
            pl.BlockSpec(memory_space=pl.ANY),
        ],
        out_specs=pl.BlockSpec((TM, D), lambda t, *_: (t, 0)),
        scratch_shapes=[
            pltpu.VMEM((2, DFF2, D), jnp.float32),
            pltpu.VMEM((2, D, DFF2 // 2), jnp.float32),
            pltpu.VMEM((DFF2, D), jnp.bfloat16),
            pltpu.VMEM((D, DFF2 // 2), jnp.bfloat16),
            pltpu.SemaphoreType.DMA((2, 4)),
        ],
    )
    return pl.pallas_call(
        _gmm_kernel,
        grid_spec=grid_spec,
        out_shape=jax.ShapeDtypeStruct((PAD, D), jnp.float32),
    )(te, tv, sl, pf, en, fi, buf, w_up, w_down)


def _combine_gather(h3, dk0, dk1):
    mesh = plsc.VectorSubcoreMesh(core_axis_name="c", subcore_axis_name="s")

    @functools.partial(
        pl.kernel,
        mesh=mesh,
        out_type=(
            jax.ShapeDtypeStruct((S, D), jnp.float32),
            jax.ShapeDtypeStruct((S, D), jnp.float32),
        ),
        scratch_types=[
            pltpu.VMEM((TW, D), jnp.float32),
            pltpu.VMEM((TW,), jnp.int32),
            pltpu.SemaphoreType.DMA,
        ],
    )
    def body(h3_hbm, dk0_hbm, dk1_hbm, g0_hbm, g1_hbm, rows_v, idx_v, sem):
        wid = lax.axis_index("s") * _NC + lax.axis_index("c")
        base = wid * TW
        pltpu.sync_copy(dk0_hbm.at[pl.ds(base, TW)], idx_v)
        pltpu.async_copy(h3_hbm.at[idx_v], rows_v, sem).wait()
        pltpu.sync_copy(rows_v, g0_hbm.at[pl.ds(base, TW)])
        pltpu.sync_copy(dk1_hbm.at[pl.ds(base, TW)], idx_v)
        pltpu.async_copy(h3_hbm.at[idx_v], rows_v, sem).wait()
        pltpu.sync_copy(rows_v, g1_hbm.at[pl.ds(base, TW)])

    return body(h3, dk0, dk1)


def _proj_kernel(g0_ref, g1_ref, cw0_ref, cw1_ref, wp_ref, out_ref):
    hc = cw0_ref[...] * g0_ref[...] + cw1_ref[...] * g1_ref[...]
    out_ref[...] = lax.dot_general(hc.astype(jnp.bfloat16),
                                   wp_ref[0].astype(jnp.bfloat16),
                                   (((1,), (1,)), ((), ())),
                                   preferred_element_type=jnp.float32)


def _run_proj(g0, g1, cw0, cw1, w_proj):
    TN = 256
    vocab = w_proj.shape[1]
    return pl.pallas_call(
        _proj_kernel,
        grid=(S // TN,),
        in_specs=[
            pl.BlockSpec((TN, D), lambda t: (t, 0)),
            pl.BlockSpec((TN, D), lambda t: (t, 0)),
            pl.BlockSpec((TN, 1), lambda t: (t, 0)),
            pl.BlockSpec((TN, 1), lambda t: (t, 0)),
            pl.BlockSpec((1, vocab, D), lambda t: (0, 0, 0)),
        ],
        out_specs=pl.BlockSpec((TN, vocab), lambda t: (t, 0)),
        out_shape=jax.ShapeDtypeStruct((S, vocab), jnp.float32),
    )(g0, g1, cw0, cw1, w_proj)


def kernel(x, w_router, w_up, w_down, w_proj):
    B, SS, DD = x.shape
    x2 = x.reshape(SS, DD)
    wrp = jnp.pad(w_router, ((0, 128 - NE), (0, 0)))

    cw0, cw1, dk0, dk1, te, tv, sl, pf, en, fi = _run_router(x2, wrp)
    buf = _dispatch_scatter(x2, dk0.reshape(S), dk1.reshape(S))
    h3 = _run_gmm(buf, w_up, w_down, te.reshape(NTPAD), tv.reshape(NTPAD),
                  sl.reshape(NTPAD), pf.reshape(NTPAD), en.reshape(NTPAD),
                  fi.reshape(NTPAD))
    g0, g1 = _combine_gather(h3, dk0.reshape(S), dk1.reshape(S))
    out = _run_proj(g0, g1, cw0, cw1, w_proj)
    return out.reshape(B, SS, w_proj.shape[1])


# speedup vs baseline: 4.5392x; 1.0022x over previous
import functools

import jax
import jax.numpy as jnp
from jax import lax
from jax.experimental import pallas as pl
from jax.experimental.pallas import tpu as pltpu
from jax.experimental.pallas import tpu_sc as plsc

NE = 8
S = 2048
D = 768
DFF2 = 4096
VPAD = 1024
TM = 128
NT = 40
NTPAD = 64
PAD = NT * TM
NP = 2 * S

_NC, _NS = 2, 16
NW = _NC * _NS
TW = S // NW


def _router_kernel(x_ref, wr_ref, cw0_ref, cw1_ref, dkc_ref, meta_ref):
    x = x_ref[...]
    scores = lax.dot_general(x, wr_ref[...], (((1,), (1,)), ((), ())),
                             preferred_element_type=jnp.float32)
    col = lax.broadcasted_iota(jnp.int32, scores.shape, 1)
    neg = jnp.float32(-3e38)
    sm = jnp.where(col < NE, scores, neg)
    m1 = jnp.max(sm, axis=1, keepdims=True)
    i1 = jnp.min(jnp.where(sm == m1, col, 9999), axis=1, keepdims=True)
    s2 = jnp.where(col == i1, neg, sm)
    m2 = jnp.max(s2, axis=1, keepdims=True)
    i2 = jnp.min(jnp.where(s2 == m2, col, 9999), axis=1, keepdims=True)
    cw0 = 1.0 / (1.0 + jnp.exp(m2 - m1))
    cw0_ref[...] = cw0
    cw1_ref[...] = 1.0 - cw0

    oh1 = (col == i1).astype(jnp.float32)
    oh2 = (col == i2).astype(jnp.float32)
    cnt = oh1 + oh2
    r = lax.broadcasted_iota(jnp.int32, (TM, TM), 0)
    c = lax.broadcasted_iota(jnp.int32, (TM, TM), 1)
    tri = (r > c).astype(jnp.bfloat16)
    blocks = []
    carry = jnp.zeros((1, 128), jnp.float32)
    for b in range(S // TM):
        cb = cnt[b * TM:(b + 1) * TM]
        blocks.append(lax.dot_general(tri, cb.astype(jnp.bfloat16),
                                      (((1,), (0,)), ((), ())),
                                      preferred_element_type=jnp.float32)
                      + carry)
        carry = carry + jnp.sum(cb, axis=0, keepdims=True)
    C = jnp.concatenate(blocks, axis=0)
    counts = carry
    rank1 = jnp.sum(C * oh1, axis=1, keepdims=True)
    rank2 = jnp.sum(C * oh2, axis=1, keepdims=True)

    ci = counts.astype(jnp.int32)
    pc = ((ci + TM - 1) // TM) * TM
    er = lax.broadcasted_iota(jnp.int32, (128, 128), 0)
    ec = lax.broadcasted_iota(jnp.int32, (128, 128), 1)
    triL = (er < ec).astype(jnp.float32)
    poff = lax.dot_general(pc.astype(jnp.float32), triL,
                           (((1,), (0,)), ((), ())),
                           preferred_element_type=jnp.float32)
    dk0 = jnp.sum(oh1 * poff, axis=1, keepdims=True) + rank1
    dk1 = jnp.sum(oh2 * poff, axis=1, keepdims=True) + rank2
    dkc_ref[...] = jnp.concatenate([dk0, dk1], axis=0).astype(jnp.int32)

    ts = lax.broadcasted_iota(jnp.int32, (NTPAD, 128), 0) * TM
    poffb = poff.astype(jnp.int32)
    ecol = lax.broadcasted_iota(jnp.int32, (NTPAD, 128), 1)
    used = (pc > 0).astype(jnp.int32)
    started = used * (poffb <= ts)
    k_ord = jnp.maximum(jnp.sum(started, axis=1, keepdims=True) - 1, 0)
    ordmat = lax.dot_general(used.astype(jnp.float32), triL,
                             (((1,), (0,)), ((), ())),
                             preferred_element_type=jnp.float32)
    ord_of = (ordmat.astype(jnp.int32) + used - 1)
    n_used = jnp.sum(used)
    sel_cur = ((ord_of == k_ord) & (used > 0)).astype(jnp.int32)
    te = jnp.sum(sel_cur * ecol, axis=1, keepdims=True)
    cnt_sel = jnp.sum(sel_cur * ci, axis=1, keepdims=True)
    poff_sel = jnp.sum(sel_cur * poffb, axis=1, keepdims=True)
    tstart = lax.broadcasted_iota(jnp.int32, (NTPAD, 1), 0) * TM
    tv = jnp.clip(cnt_sel - (tstart - poff_sel), 0, TM)
    sel_nxt = ((ord_of == k_ord + 1) & (used > 0)).astype(jnp.int32)
    enext = jnp.sum(sel_nxt * ecol, axis=1, keepdims=True)
    first = (tstart == poff_sel).astype(jnp.int32)
    pref = first * (k_ord + 1 < n_used).astype(jnp.int32)
    slot = k_ord & 1
    meta_ref[...] = jnp.concatenate(
        [te, tv, slot, pref, enext, first, first, first], axis=1)


def _run_router(x2, wrp):
    f32 = jnp.float32
    i32 = jnp.int32
    return pl.pallas_call(
        _router_kernel,
        out_shape=(
            jax.ShapeDtypeStruct((S, 1), f32),
            jax.ShapeDtypeStruct((S, 1), f32),
            jax.ShapeDtypeStruct((NP, 1), i32),
            jax.ShapeDtypeStruct((NTPAD, 8), i32),
        ),
    )(x2, wrp)


def _dispatch_scatter(x2, dkc):
    mesh = plsc.VectorSubcoreMesh(core_axis_name="c", subcore_axis_name="s")

    @functools.partial(
        pl.kernel,
        mesh=mesh,
        out_type=jax.ShapeDtypeStruct((PAD, D), jnp.float32),
        scratch_types=[
            pltpu.VMEM((TW, D), jnp.float32),
            pltpu.VMEM((TW,), jnp.int32),
            pltpu.VMEM((TW,), jnp.int32),
            pltpu.SemaphoreType.DMA,
        ],
    )
    def body(x_hbm, dkc_hbm, buf_hbm, rows_v, idx0_v, idx1_v, sem):
        wid = lax.axis_index("s") * _NC + lax.axis_index("c")
        base = wid * TW
        pltpu.sync_copy(x_hbm.at[pl.ds(base, TW)], rows_v)
        pltpu.sync_copy(dkc_hbm.at[pl.ds(base, TW)], idx0_v)
        pltpu.sync_copy(dkc_hbm.at[pl.ds(S + base, TW)], idx1_v)
        pltpu.async_copy(rows_v, buf_hbm.at[idx0_v], sem).wait()
        pltpu.async_copy(rows_v, buf_hbm.at[idx1_v], sem).wait()

    return body(x2, dkc)


def _wdma(wup_hbm, wdn_hbm, wup_v, wdn_v, sems, e, s):
    h = DFF2 // 2
    q = D // 2
    return (
        pltpu.make_async_copy(wup_hbm.at[e, pl.ds(0, h)],
                              wup_v.at[s, pl.ds(0, h)], sems.at[s, 0]),
        pltpu.make_async_copy(wup_hbm.at[e, pl.ds(h, h)],
                              wup_v.at[s, pl.ds(h, h)], sems.at[s, 1]),
        pltpu.make_async_copy(wdn_hbm.at[e, pl.ds(0, q)],
                              wdn_v.at[s, pl.ds(0, q)], sems.at[s, 2]),
        pltpu.make_async_copy(wdn_hbm.at[e, pl.ds(q, q)],
                              wdn_v.at[s, pl.ds(q, q)], sems.at[s, 3]),
    )


def _gmm_kernel(meta_ref, x_ref, wup_hbm, wdn_hbm, out_ref,
                wup_v, wdn_v, wupb_ref, wdnb_ref, sems):
    t = pl.program_id(0)
    e = meta_ref[t, 0]
    s = meta_ref[t, 2]
    first = meta_ref[t, 5] == 1

    @pl.when(t == 0)
    def _start_first():
        for cp in _wdma(wup_hbm, wdn_hbm, wup_v, wdn_v, sems, e, s):
            cp.start()

    @pl.when(meta_ref[t, 3] == 1)
    def _start_next():
        for cp in _wdma(wup_hbm, wdn_hbm, wup_v, wdn_v, sems,
                        meta_ref[t, 4], 1 - s):
            cp.start()

    @pl.when(first | (t == 0))
    def _wait_and_cast():
        for cp in _wdma(wup_hbm, wdn_hbm, wup_v, wdn_v, sems, e, s):
            cp.wait()
        wupb_ref[...] = wup_v[s].astype(jnp.bfloat16)
        wdnb_ref[...] = wdn_v[s].astype(jnp.bfloat16)

    v = meta_ref[t, 1]
    x = x_ref[...]
    rows = lax.broadcasted_iota(jnp.int32, (TM, 1), 0)
    x = jnp.where(rows < v, x, 0.0)
    xb = x.astype(jnp.bfloat16)
    h1 = lax.dot_general(xb, wupb_ref[...], (((1,), (1,)), ((), ())),
                         preferred_element_type=jnp.float32)
    h1 = h1.astype(jnp.bfloat16).astype(jnp.float32)
    a = h1[:, :DFF2 // 2]
    g = h1[:, DFF2 // 2:]
    sw = a * (g * (1.0 / (1.0 + jnp.exp(-g))))
    h2 = lax.dot_general(sw.astype(jnp.bfloat16), wdnb_ref[...],
                         (((1,), (1,)), ((), ())),
                         preferred_element_type=jnp.float32)
    h2 = h2.astype(jnp.bfloat16).astype(jnp.float32)
    out_ref[...] = h2 + x


def _run_gmm(buf, w_up, w_down, meta):
    grid_spec = pltpu.PrefetchScalarGridSpec(
        num_scalar_prefetch=1,
        grid=(NT,),
        in_specs=[
            pl.BlockSpec((TM, D), lambda t, *_: (t, 0)),
            pl.BlockSpec(memory_space=pl.ANY),
            pl.BlockSpec(memory_space=pl.ANY),
        ],
        out_specs=pl.BlockSpec((TM, D), lambda t, *_: (t, 0)),
        scratch_shapes=[
            pltpu.VMEM((2, DFF2, D), jnp.float32),
            pltpu.VMEM((2, D, DFF2 // 2), jnp.float32),
            pltpu.VMEM((DFF2, D), jnp.bfloat16),
            pltpu.VMEM((D, DFF2 // 2), jnp.bfloat16),
            pltpu.SemaphoreType.DMA((2, 4)),
        ],
    )
    return pl.pallas_call(
        _gmm_kernel,
        grid_spec=grid_spec,
        out_shape=jax.ShapeDtypeStruct((PAD, D), jnp.float32),
    )(meta, buf, w_up, w_down)


def _combine_gather(h3, dkc):
    mesh = plsc.VectorSubcoreMesh(core_axis_name="c", subcore_axis_name="s")

    @functools.partial(
        pl.kernel,
        mesh=mesh,
        out_type=(
            jax.ShapeDtypeStruct((S, D), jnp.float32),
            jax.ShapeDtypeStruct((S, D), jnp.float32),
        ),
        scratch_types=[
            pltpu.VMEM((TW, D), jnp.float32),
            pltpu.VMEM((TW,), jnp.int32),
            pltpu.SemaphoreType.DMA,
        ],
    )
    def body(h3_hbm, dkc_hbm, g0_hbm, g1_hbm, rows_v, idx_v, sem):
        wid = lax.axis_index("s") * _NC + lax.axis_index("c")
        base = wid * TW
        pltpu.sync_copy(dkc_hbm.at[pl.ds(base, TW)], idx_v)
        pltpu.async_copy(h3_hbm.at[idx_v], rows_v, sem).wait()
        pltpu.sync_copy(rows_v, g0_hbm.at[pl.ds(base, TW)])
        pltpu.sync_copy(dkc_hbm.at[pl.ds(S + base, TW)], idx_v)
        pltpu.async_copy(h3_hbm.at[idx_v], rows_v, sem).wait()
        pltpu.sync_copy(rows_v, g1_hbm.at[pl.ds(base, TW)])

    return body(h3, dkc)


def _proj_kernel(g0_ref, g1_ref, cw0_ref, cw1_ref, wp_ref, out_ref):
    hc = cw0_ref[...] * g0_ref[...] + cw1_ref[...] * g1_ref[...]
    out_ref[...] = lax.dot_general(hc.astype(jnp.bfloat16),
                                   wp_ref[0].astype(jnp.bfloat16),
                                   (((1,), (1,)), ((), ())),
                                   preferred_element_type=jnp.float32)


def _run_proj(g0, g1, cw0, cw1, w_proj):
    TN = 256
    vocab = w_proj.shape[1]
    return pl.pallas_call(
        _proj_kernel,
        grid=(S // TN,),
        in_specs=[
            pl.BlockSpec((TN, D), lambda t: (t, 0)),
            pl.BlockSpec((TN, D), lambda t: (t, 0)),
            pl.BlockSpec((TN, 1), lambda t: (t, 0)),
            pl.BlockSpec((TN, 1), lambda t: (t, 0)),
            pl.BlockSpec((1, vocab, D), lambda t: (0, 0, 0)),
        ],
        out_specs=pl.BlockSpec((TN, vocab), lambda t: (t, 0)),
        out_shape=jax.ShapeDtypeStruct((S, vocab), jnp.float32),
    )(g0, g1, cw0, cw1, w_proj)


def kernel(x, w_router, w_up, w_down, w_proj):
    B, SS, DD = x.shape
    x2 = x.reshape(SS, DD)
    wrp = jnp.pad(w_router, ((0, 128 - NE), (0, 0)))

    cw0, cw1, dkc, meta = _run_router(x2, wrp)
    dkc = dkc.reshape(NP)
    buf = _dispatch_scatter(x2, dkc)
    h3 = _run_gmm(buf, w_up, w_down, meta)
    g0, g1 = _combine_gather(h3, dkc)
    out = _run_proj(g0, g1, cw0, cw1, w_proj)
    return out.reshape(B, SS, w_proj.shape[1])


# speedup vs baseline: 4.5572x; 1.0040x over previous
import functools

import jax
import jax.numpy as jnp
from jax import lax
from jax.experimental import pallas as pl
from jax.experimental.pallas import tpu as pltpu
from jax.experimental.pallas import tpu_sc as plsc

NE = 8
S = 2048
D = 768
DFF2 = 4096
TM = 128
NT = 40
NTPAD = 64
PAD = NT * TM
NP = 2 * S

_NC, _NS = 2, 16
NW = _NC * _NS
TW = S // NW


def _router_kernel(x_ref, wr_ref, cw0_ref, cw1_ref, dkc_ref, meta_ref):
    x = x_ref[...]
    scores = lax.dot_general(x, wr_ref[...], (((1,), (1,)), ((), ())),
                             preferred_element_type=jnp.float32)
    col = lax.broadcasted_iota(jnp.int32, scores.shape, 1)
    neg = jnp.float32(-3e38)
    sm = jnp.where(col < NE, scores, neg)
    m1 = jnp.max(sm, axis=1, keepdims=True)
    i1 = jnp.min(jnp.where(sm == m1, col, 9999), axis=1, keepdims=True)
    s2 = jnp.where(col == i1, neg, sm)
    m2 = jnp.max(s2, axis=1, keepdims=True)
    i2 = jnp.min(jnp.where(s2 == m2, col, 9999), axis=1, keepdims=True)
    cw0 = 1.0 / (1.0 + jnp.exp(m2 - m1))
    cw0_ref[...] = cw0
    cw1_ref[...] = 1.0 - cw0

    oh1 = (col == i1).astype(jnp.float32)
    oh2 = (col == i2).astype(jnp.float32)
    cnt = oh1 + oh2
    r = lax.broadcasted_iota(jnp.int32, (TM, TM), 0)
    c = lax.broadcasted_iota(jnp.int32, (TM, TM), 1)
    tri = (r > c).astype(jnp.bfloat16)
    blocks = []
    carry = jnp.zeros((1, 128), jnp.float32)
    for b in range(S // TM):
        cb = cnt[b * TM:(b + 1) * TM]
        blocks.append(lax.dot_general(tri, cb.astype(jnp.bfloat16),
                                      (((1,), (0,)), ((), ())),
                                      preferred_element_type=jnp.float32)
                      + carry)
        carry = carry + jnp.sum(cb, axis=0, keepdims=True)
    C = jnp.concatenate(blocks, axis=0)
    counts = carry
    rank1 = jnp.sum(C * oh1, axis=1, keepdims=True)
    rank2 = jnp.sum(C * oh2, axis=1, keepdims=True)

    ci = counts.astype(jnp.int32)
    pc = ((ci + TM - 1) // TM) * TM
    er = lax.broadcasted_iota(jnp.int32, (128, 128), 0)
    ec = lax.broadcasted_iota(jnp.int32, (128, 128), 1)
    triL = (er < ec).astype(jnp.float32)
    poff = lax.dot_general(pc.astype(jnp.float32), triL,
                           (((1,), (0,)), ((), ())),
                           preferred_element_type=jnp.float32)
    dk0 = jnp.sum(oh1 * poff, axis=1, keepdims=True) + rank1
    dk1 = jnp.sum(oh2 * poff, axis=1, keepdims=True) + rank2
    dkc_ref[...] = jnp.concatenate([dk0, dk1], axis=0).astype(jnp.int32)

    ts = lax.broadcasted_iota(jnp.int32, (NTPAD, 128), 0) * TM
    poffb = poff.astype(jnp.int32)
    ecol = lax.broadcasted_iota(jnp.int32, (NTPAD, 128), 1)
    used = (pc > 0).astype(jnp.int32)
    started = used * (poffb <= ts)
    k_ord = jnp.maximum(jnp.sum(started, axis=1, keepdims=True) - 1, 0)
    ordmat = lax.dot_general(used.astype(jnp.float32), triL,
                             (((1,), (0,)), ((), ())),
                             preferred_element_type=jnp.float32)
    ord_of = (ordmat.astype(jnp.int32) + used - 1)
    n_used = jnp.sum(used)
    sel_cur = ((ord_of == k_ord) & (used > 0)).astype(jnp.int32)
    te = jnp.sum(sel_cur * ecol, axis=1, keepdims=True)
    cnt_sel = jnp.sum(sel_cur * ci, axis=1, keepdims=True)
    poff_sel = jnp.sum(sel_cur * poffb, axis=1, keepdims=True)
    tstart = lax.broadcasted_iota(jnp.int32, (NTPAD, 1), 0) * TM
    tv = jnp.clip(cnt_sel - (tstart - poff_sel), 0, TM)
    sel_nxt = ((ord_of == k_ord + 1) & (used > 0)).astype(jnp.int32)
    enext = jnp.sum(sel_nxt * ecol, axis=1, keepdims=True)
    first = (tstart == poff_sel).astype(jnp.int32)
    pref = first * (k_ord + 1 < n_used).astype(jnp.int32)
    slot = k_ord & 1
    meta_ref[...] = jnp.concatenate(
        [te, tv, slot, pref, enext, first, first, first], axis=1)


def _run_router(x2, wrp):
    f32 = jnp.float32
    i32 = jnp.int32
    return pl.pallas_call(
        _router_kernel,
        out_shape=(
            jax.ShapeDtypeStruct((S, 1), f32),
            jax.ShapeDtypeStruct((S, 1), f32),
            jax.ShapeDtypeStruct((NP, 1), i32),
            jax.ShapeDtypeStruct((NTPAD, 8), i32),
        ),
    )(x2, wrp)


def _dispatch_scatter(x2, dkc):
    mesh = plsc.VectorSubcoreMesh(core_axis_name="c", subcore_axis_name="s")

    @functools.partial(
        pl.kernel,
        mesh=mesh,
        out_type=jax.ShapeDtypeStruct((PAD, D), jnp.float32),
        scratch_types=[
            pltpu.VMEM((TW, D), jnp.float32),
            pltpu.VMEM((TW,), jnp.int32),
            pltpu.VMEM((TW,), jnp.int32),
            pltpu.SemaphoreType.DMA,
        ],
    )
    def body(x_hbm, dkc_hbm, buf_hbm, rows_v, idx0_v, idx1_v, sem):
        wid = lax.axis_index("s") * _NC + lax.axis_index("c")
        base = wid * TW
        pltpu.sync_copy(x_hbm.at[pl.ds(base, TW)], rows_v)
        pltpu.sync_copy(dkc_hbm.at[pl.ds(base, TW)], idx0_v)
        pltpu.sync_copy(dkc_hbm.at[pl.ds(S + base, TW)], idx1_v)
        pltpu.async_copy(rows_v, buf_hbm.at[idx0_v], sem).wait()
        pltpu.async_copy(rows_v, buf_hbm.at[idx1_v], sem).wait()

    return body(x2, dkc)


def _wdma(wup_hbm, wdn_hbm, wup_v, wdn_v, sems, e, s):
    h = DFF2 // 2
    q = D // 2
    return (
        pltpu.make_async_copy(wup_hbm.at[e, pl.ds(0, h)],
                              wup_v.at[s, pl.ds(0, h)], sems.at[s, 0]),
        pltpu.make_async_copy(wup_hbm.at[e, pl.ds(h, h)],
                              wup_v.at[s, pl.ds(h, h)], sems.at[s, 1]),
        pltpu.make_async_copy(wdn_hbm.at[e, pl.ds(0, q)],
                              wdn_v.at[s, pl.ds(0, q)], sems.at[s, 2]),
        pltpu.make_async_copy(wdn_hbm.at[e, pl.ds(q, q)],
                              wdn_v.at[s, pl.ds(q, q)], sems.at[s, 3]),
    )


def _gmm_kernel(meta_ref, x_ref, wup_hbm, wdn_hbm, out_ref,
                wup_v, wdn_v, wupb_ref, wdnb_ref, sems):
    t = pl.program_id(0)
    e = meta_ref[t, 0]
    s = meta_ref[t, 2]
    first = meta_ref[t, 5] == 1

    @pl.when(t == 0)
    def _start_first():
        for cp in _wdma(wup_hbm, wdn_hbm, wup_v, wdn_v, sems, e, s):
            cp.start()

    @pl.when(meta_ref[t, 3] == 1)
    def _start_next():
        for cp in _wdma(wup_hbm, wdn_hbm, wup_v, wdn_v, sems,
                        meta_ref[t, 4], 1 - s):
            cp.start()

    @pl.when(first | (t == 0))
    def _wait_and_cast():
        for cp in _wdma(wup_hbm, wdn_hbm, wup_v, wdn_v, sems, e, s):
            cp.wait()
        wupb_ref[...] = wup_v[s].astype(jnp.bfloat16)
        wdnb_ref[...] = wdn_v[s].astype(jnp.bfloat16)

    v = meta_ref[t, 1]
    x = x_ref[...]
    rows = lax.broadcasted_iota(jnp.int32, (TM, 1), 0)
    x = jnp.where(rows < v, x, 0.0)
    xb = x.astype(jnp.bfloat16)
    h1 = lax.dot_general(xb, wupb_ref[...], (((1,), (1,)), ((), ())),
                         preferred_element_type=jnp.float32)
    h1 = h1.astype(jnp.bfloat16).astype(jnp.float32)
    a = h1[:, :DFF2 // 2]
    g = h1[:, DFF2 // 2:]
    sw = a * (g * (1.0 / (1.0 + jnp.exp(-g))))
    h2 = lax.dot_general(sw.astype(jnp.bfloat16), wdnb_ref[...],
                         (((1,), (1,)), ((), ())),
                         preferred_element_type=jnp.float32)
    h2 = h2.astype(jnp.bfloat16).astype(jnp.float32)
    out_ref[...] = h2 + x


def _run_gmm(buf, w_up, w_down, meta):
    grid_spec = pltpu.PrefetchScalarGridSpec(
        num_scalar_prefetch=1,
        grid=(NT,),
        in_specs=[
            pl.BlockSpec((TM, D), lambda t, *_: (t, 0)),
            pl.BlockSpec(memory_space=pl.ANY),
            pl.BlockSpec(memory_space=pl.ANY),
        ],
        out_specs=pl.BlockSpec((TM, D), lambda t, *_: (t, 0)),
        scratch_shapes=[
            pltpu.VMEM((2, DFF2, D), jnp.float32),
            pltpu.VMEM((2, D, DFF2 // 2), jnp.float32),
            pltpu.VMEM((DFF2, D), jnp.bfloat16),
            pltpu.VMEM((D, DFF2 // 2), jnp.bfloat16),
            pltpu.SemaphoreType.DMA((2, 4)),
        ],
    )
    return pl.pallas_call(
        _gmm_kernel,
        grid_spec=grid_spec,
        out_shape=jax.ShapeDtypeStruct((PAD, D), jnp.float32),
    )(meta, buf, w_up, w_down)


def _combine_gather(h3, dkc):
    mesh = plsc.VectorSubcoreMesh(core_axis_name="c", subcore_axis_name="s")

    @functools.partial(
        pl.kernel,
        mesh=mesh,
        out_type=(
            jax.ShapeDtypeStruct((S, D), jnp.float32),
            jax.ShapeDtypeStruct((S, D), jnp.float32),
        ),
        scratch_types=[
            pltpu.VMEM((TW, D), jnp.float32),
            pltpu.VMEM((TW,), jnp.int32),
            pltpu.SemaphoreType.DMA,
        ],
    )
    def body(h3_hbm, dkc_hbm, g0_hbm, g1_hbm, rows_v, idx_v, sem):
        wid = lax.axis_index("s") * _NC + lax.axis_index("c")
        base = wid * TW
        pltpu.sync_copy(dkc_hbm.at[pl.ds(base, TW)], idx_v)
        pltpu.async_copy(h3_hbm.at[idx_v], rows_v, sem).wait()
        pltpu.sync_copy(rows_v, g0_hbm.at[pl.ds(base, TW)])
        pltpu.sync_copy(dkc_hbm.at[pl.ds(S + base, TW)], idx_v)
        pltpu.async_copy(h3_hbm.at[idx_v], rows_v, sem).wait()
        pltpu.sync_copy(rows_v, g1_hbm.at[pl.ds(base, TW)])

    return body(h3, dkc)


def _proj_kernel(g0_ref, g1_ref, cw0_ref, cw1_ref, wp_ref, out_ref):
    hc = cw0_ref[...] * g0_ref[...] + cw1_ref[...] * g1_ref[...]
    out_ref[...] = lax.dot_general(hc.astype(jnp.bfloat16),
                                   wp_ref[0].astype(jnp.bfloat16),
                                   (((1,), (1,)), ((), ())),
                                   preferred_element_type=jnp.float32)


def _run_proj(g0, g1, cw0, cw1, w_proj):
    TN = 256
    vocab = w_proj.shape[1]
    return pl.pallas_call(
        _proj_kernel,
        grid=(S // TN,),
        in_specs=[
            pl.BlockSpec((TN, D), lambda t: (t, 0)),
            pl.BlockSpec((TN, D), lambda t: (t, 0)),
            pl.BlockSpec((TN, 1), lambda t: (t, 0)),
            pl.BlockSpec((TN, 1), lambda t: (t, 0)),
            pl.BlockSpec((1, vocab, D), lambda t: (0, 0, 0)),
        ],
        out_specs=pl.BlockSpec((TN, vocab), lambda t: (t, 0)),
        out_shape=jax.ShapeDtypeStruct((S, vocab), jnp.float32),
    )(g0, g1, cw0, cw1, w_proj)


def kernel(x, w_router, w_up, w_down, w_proj):
    B, SS, DD = x.shape
    x2 = x.reshape(SS, DD)
    wrp = jnp.pad(w_router, ((0, 128 - NE), (0, 0)))

    cw0, cw1, dkc, meta = _run_router(x2, wrp)
    dkc = dkc.reshape(NP)
    buf = _dispatch_scatter(x2, dkc)
    h3 = _run_gmm(buf, w_up, w_down, meta)
    g0, g1 = _combine_gather(h3, dkc)
    out = _run_proj(g0, g1, cw0, cw1, w_proj)
    return out.reshape(B, SS, w_proj.shape[1])
